# baseline (speedup 1.0000x reference)
import functools

import jax
import jax.numpy as jnp
from jax.experimental import pallas as pl
from jax.experimental.pallas import tpu as pltpu

CP = 128
VMEM = 32 * 1024 * 1024
SK, SG, SKP = 7, 32, 224


def _rup(v, m):
    return (v + m - 1) // m * m


def _pick_tr(ho, wo, target=256, cap=2048):
    best = 1
    for d in range(1, ho + 1):
        if ho % d:
            continue
        if d * wo > cap and best > 1:
            break
        if d != ho and (d * wo) % 8:
            continue
        best = d
        if d * wo >= target:
            break
    return best


def _squeeze_w(w, cin, k=3):
    kk = k * k * cin
    wc = w.reshape(k * k, CP, CP)[:, :cin, :].reshape(kk, CP)
    return jnp.pad(wc, ((0, _rup(kk, CP) - kk), (0, 0)))


def _conv_body(*refs, k, stride, wo, tr, cin, kp, relu, has_ds, has_res):
    it = iter(refs)
    x_ref = next(it)
    w_ref = next(it)
    b_ref = next(it)
    wd_ref = next(it) if has_ds else None
    bd_ref = next(it) if has_ds else None
    res_ref = next(it) if has_res else None
    o_ref = next(it)
    od_ref = next(it) if has_ds else None
    p_ref = next(it)
    dp_ref = next(it) if has_ds else None

    r = pl.program_id(1)
    row0 = pl.multiple_of(r * tr * stride, tr * stride)
    kk = k * k * cin
    if kp > kk:
        p_ref[:, kk:] = jnp.zeros((tr * wo, kp - kk), jnp.bfloat16)
    if has_ds:
        dp_ref[:, cin:] = jnp.zeros((tr * wo, CP - cin), jnp.bfloat16)

    for t in range(tr):
        for kh in range(k):
            row = x_ref[0, row0 + t * stride + kh]
            for kw in range(k):
                if stride == 1:
                    sl = row[kw:kw + wo, 0:cin]
                else:
                    q, hb = divmod(kw, 2)
                    sl = row[q:q + wo, hb * CP:hb * CP + cin]
                c0 = (kh * k + kw) * cin
                p_ref[t * wo:(t + 1) * wo, c0:c0 + cin] = sl
            if has_ds and kh == 1:
                dsl = row[0:wo, CP:CP + cin]
                dp_ref[t * wo:(t + 1) * wo, 0:cin] = dsl

    y = jnp.dot(p_ref[...], w_ref[...],
                preferred_element_type=jnp.float32) + b_ref[...]
    if has_res:
        y = y + res_ref[0].astype(jnp.float32)
    if relu:
        y = jnp.maximum(y, 0.0)
    o_ref[0] = y.astype(o_ref.dtype)

    if has_ds:
        yd = jnp.dot(dp_ref[...], wd_ref[...],
                     preferred_element_type=jnp.float32) + bd_ref[...]
        od_ref[0] = yd.astype(od_ref.dtype)


def _stem_body(x_ref, w_ref, b_ref, o_ref, p_ref, *, wo, tr):
    r = pl.program_id(1)
    row0 = pl.multiple_of(r * tr * 2, tr * 2)
    for t in range(tr):
        for kh in range(SK):
            p_ref[t * wo:(t + 1) * wo, kh * SG:(kh + 1) * SG] = \
                x_ref[0, row0 + 2 * t + kh]
    y = jnp.dot(p_ref[...], w_ref[...],
                preferred_element_type=jnp.float32) + b_ref[...]
    o_ref[0] = jnp.maximum(y, 0.0).astype(o_ref.dtype)


def _pool_body(x_ref, o_ref, *, wo, tr):
    r = pl.program_id(1)
    row0 = pl.multiple_of(r * tr * 2, tr * 2)
    for t in range(tr):
        acc = None
        for kh in range(3):
            row = x_ref[0, row0 + 2 * t + kh]
            for kw in range(3):
                q, hb = divmod(kw, 2)
                sl = row[q:q + wo, hb * CP:(hb + 1) * CP]
                acc = sl if acc is None else jnp.maximum(acc, sl)
        o_ref[0, t * wo:(t + 1) * wo, :] = acc


def _conv(x, w, b, *, stride, relu, cin, residual=None, wd=None, bd=None):
    n, h, wdim, _ = x.shape
    k, pad = 3, 1
    ho = (h + 2 * pad - k) // stride + 1
    wo = (wdim + 2 * pad - k) // stride + 1
    has_ds = wd is not None
    has_res = residual is not None

    xp = jnp.pad(x, ((0, 0), (pad, pad), (pad, pad), (0, 0)))
    hp, wp = h + 2 * pad, wdim + 2 * pad
    if stride == 2:
        xin = xp.reshape(n, hp, wp // 2, 2 * CP)
    else:
        xin = xp
    wrow, crow = xin.shape[2], xin.shape[3]

    tr = _pick_tr(ho, wo)
    kp = _rup(k * k * cin, CP)

    in_specs = [
        pl.BlockSpec((1, hp, wrow, crow), lambda nn, rr: (nn, 0, 0, 0)),
        pl.BlockSpec(w.shape, lambda nn, rr: (0, 0)),
        pl.BlockSpec(b.shape, lambda nn, rr: (0, 0)),
    ]
    inputs = [xin, w, b]
    if has_ds:
        in_specs += [pl.BlockSpec(wd.shape, lambda nn, rr: (0, 0)),
                     pl.BlockSpec(bd.shape, lambda nn, rr: (0, 0))]
        inputs += [wd, bd]
    if has_res:
        in_specs.append(pl.BlockSpec((1, tr * wo, CP),
                                     lambda nn, rr: (nn, rr, 0)))
        inputs.append(residual.reshape(n, ho * wo, CP))

    out_spec = pl.BlockSpec((1, tr * wo, CP), lambda nn, rr: (nn, rr, 0))
    out_shape = jax.ShapeDtypeStruct((n, ho * wo, CP), jnp.bfloat16)
    scratch = [pltpu.VMEM((tr * wo, kp), jnp.bfloat16)]
    if has_ds:
        scratch.append(pltpu.VMEM((tr * wo, CP), jnp.bfloat16))

    flops = 2 * n * ho * wo * kp * CP + (2 * n * ho * wo * CP * CP
                                         if has_ds else 0)
    bytes_acc = (xin.size * 2 + w.size * 2
                 + n * ho * wo * CP * 2 * (1 + int(has_ds) + int(has_res)))

    outs = pl.pallas_call(
        functools.partial(_conv_body, k=k, stride=stride, wo=wo, tr=tr,
                          cin=cin, kp=kp, relu=relu, has_ds=has_ds,
                          has_res=has_res),
        out_shape=(out_shape, out_shape) if has_ds else out_shape,
        grid=(n, ho // tr),
        in_specs=in_specs,
        out_specs=(out_spec, out_spec) if has_ds else out_spec,
        scratch_shapes=scratch,
        compiler_params=pltpu.CompilerParams(
            dimension_semantics=("parallel", "parallel"),
            vmem_limit_bytes=VMEM),
        cost_estimate=pl.CostEstimate(flops=flops, transcendentals=0,
                                      bytes_accessed=bytes_acc),
    )(*inputs)

    if has_ds:
        y, yd = outs
        return y.reshape(n, ho, wo, CP), yd.reshape(n, ho, wo, CP)
    return outs.reshape(n, ho, wo, CP)


def _stem(x_nhwc, w_pad, bias):
    n, h, wdim, cin = x_nhwc.shape
    k, s, p = SK, 2, 3
    ho = (h + 2 * p - k) // s + 1
    wo = (wdim + 2 * p - k) // s + 1
    hp = h + 2 * p
    xp = jnp.pad(x_nhwc, ((0, 0), (p, p), (p, p), (0, 0))).astype(jnp.bfloat16)
    cols = [xp[:, :, kw:kw + s * wo:s, :] for kw in range(k)]
    xw = jnp.concatenate(cols, axis=-1)
    xw = jnp.pad(xw, ((0, 0), (0, 0), (0, 0), (0, SG - k * cin)))
    w_c = w_pad[:SKP]

    tr = _pick_tr(ho, wo)
    out = pl.pallas_call(
        functools.partial(_stem_body, wo=wo, tr=tr),
        out_shape=jax.ShapeDtypeStruct((n, ho * wo, CP), jnp.bfloat16),
        grid=(n, ho // tr),
        in_specs=[pl.BlockSpec((1, hp, wo, SG), lambda nn, rr: (nn, 0, 0, 0)),
                  pl.BlockSpec((SKP, CP), lambda nn, rr: (0, 0)),
                  pl.BlockSpec(bias.shape, lambda nn, rr: (0, 0))],
        out_specs=pl.BlockSpec((1, tr * wo, CP), lambda nn, rr: (nn, rr, 0)),
        scratch_shapes=[pltpu.VMEM((tr * wo, SKP), jnp.bfloat16)],
        compiler_params=pltpu.CompilerParams(
            dimension_semantics=("parallel", "parallel"),
            vmem_limit_bytes=VMEM),
        cost_estimate=pl.CostEstimate(
            flops=2 * n * ho * wo * SKP * CP, transcendentals=0,
            bytes_accessed=xw.size * 2 + SKP * CP * 2 + n * ho * wo * CP * 2),
    )(xw, w_c, bias)
    return out.reshape(n, ho, wo, CP)


def _pool(x):
    n, h, wdim, _ = x.shape
    ho, wo = h // 2, wdim // 2
    xp = jnp.pad(x, ((0, 0), (1, 1), (1, 1), (0, 0)))
    hp, wp = h + 2, wdim + 2
    xin = xp.reshape(n, hp, wp // 2, 2 * CP)
    tr = _pick_tr(ho, wo)
    out = pl.pallas_call(
        functools.partial(_pool_body, wo=wo, tr=tr),
        out_shape=jax.ShapeDtypeStruct((n, ho * wo, CP), jnp.bfloat16),
        grid=(n, ho // tr),
        in_specs=[pl.BlockSpec((1, hp, wp // 2, 2 * CP),
                               lambda nn, rr: (nn, 0, 0, 0))],
        out_specs=pl.BlockSpec((1, tr * wo, CP), lambda nn, rr: (nn, rr, 0)),
        compiler_params=pltpu.CompilerParams(
            dimension_semantics=("parallel", "parallel"),
            vmem_limit_bytes=VMEM),
    )(xin)
    return out.reshape(n, ho, wo, CP)


def _block(x, w1, b1, w2, b2, cin, cout, wd=None, bd=None):
    stride = 2 if wd is not None else 1
    if wd is not None:
        out, idn = _conv(x, _squeeze_w(w1, cin), b1, stride=stride, relu=True,
                         cin=cin, wd=wd, bd=bd)
    else:
        out = _conv(x, _squeeze_w(w1, cin), b1, stride=1, relu=True, cin=cin)
        idn = x
    return _conv(out, _squeeze_w(w2, cout), b2, stride=1, relu=True, cin=cout,
                 residual=idn)


def kernel(stem_w, stem_b, l1_w1, l1_b1, l1_w2, l1_b2, l2_w1, l2_b1, l2_w2,
           l2_b2, l2_wd, l2_bd, l3_w1, l3_b1, l3_w2, l3_b2, l3_wd, l3_bd,
           l4_w1, l4_b1, l4_w2, l4_b2, l4_wd, l4_bd, x):
    h = jnp.transpose(x.astype(jnp.float32), (0, 2, 3, 1))
    h = _stem(h, stem_w, stem_b)
    h = _pool(h)
    h = _block(h, l1_w1, l1_b1, l1_w2, l1_b2, 8, 8)
    h = _block(h, l2_w1, l2_b1, l2_w2, l2_b2, 8, 16, l2_wd, l2_bd)
    h = _block(h, l3_w1, l3_b1, l3_w2, l3_b2, 16, 32, l3_wd, l3_bd)
    h = _block(h, l4_w1, l4_b1, l4_w2, l4_b2, 32, 64, l4_wd, l4_bd)
    return {"0": jnp.transpose(h[..., :64].astype(jnp.float32), (0, 3, 1, 2))}
